# baseline (speedup 1.0000x reference)
import jax
import jax.numpy as jnp
from jax import lax
from jax.experimental import pallas as pl
from jax.experimental.pallas import tpu as pltpu

L, C, H, W = 5, 256, 100, 152
A = 3
HP, WP = H + 2, W + 2
S_PAD = -(-(HP * WP) // 128) * 128
RB = 10
NB = H // RB
BS = RB * WP
NHEAD = 16


def _rpn_kernel(x_ref, cw_ref, cb_ref, clw_ref, bbw_ref, clb_ref, bbb_ref,
                cls_ref, bbox_ref, xs_ref, w9_ref, wh_ref, hb_ref):
    lvl = pl.program_id(0)

    @pl.when(lvl == 0)
    def _prep():
        xs_ref[...] = jnp.zeros((C, S_PAD), jnp.bfloat16)
        cwb = cw_ref[...].astype(jnp.bfloat16)
        r = lax.broadcasted_iota(jnp.int32, (9 * C, 1), 0)
        for k in range(9):
            i = lax.broadcasted_iota(jnp.int32, (1, C), 1)
            sel = jnp.where(r == i * 9 + k, 1.0, 0.0).astype(jnp.bfloat16)
            w9_ref[k] = jnp.dot(cwb, sel,
                                preferred_element_type=jnp.float32
                                ).astype(jnp.bfloat16)
        wh_ref[0:A] = clw_ref[...].astype(jnp.bfloat16)
        wh_ref[A:A + 4 * A] = bbw_ref[...].astype(jnp.bfloat16)
        wh_ref[A + 4 * A:] = jnp.zeros((NHEAD - A - 4 * A, C), jnp.bfloat16)
        hb_ref[0:A] = clb_ref[...]
        hb_ref[A:A + 4 * A] = bbb_ref[...]
        hb_ref[A + 4 * A:] = jnp.zeros((NHEAD - A - 4 * A, 1), jnp.float32)

    for h in range(H):
        xs_ref[:, (h + 1) * WP + 1:(h + 1) * WP + 1 + W] = (
            x_ref[0, :, h * W:(h + 1) * W].astype(jnp.bfloat16))

    cb = cb_ref[...]
    hb = hb_ref[...]
    wh = wh_ref[...]
    for b in range(NB):
        h0 = b * RB
        acc = jnp.zeros((C, BS), jnp.float32)
        for dh in range(3):
            for dw in range(3):
                start = (h0 + dh) * WP + dw
                acc += jnp.dot(w9_ref[dh * 3 + dw],
                               xs_ref[:, start:start + BS],
                               preferred_element_type=jnp.float32)
        t = jnp.maximum(acc + cb, 0.0).astype(jnp.bfloat16)
        o = jnp.dot(wh, t, preferred_element_type=jnp.float32) + hb
        for rr in range(RB):
            row = o[:, rr * WP:rr * WP + W]
            cls_ref[0, :, h0 + rr, :] = row[:A]
            bbox_ref[0, :, h0 + rr, :] = row[A:A + 4 * A]


@jax.jit
def kernel(x, conv_w, conv_b, cls_w, cls_b, bbox_w, bbox_b):
    xf = x.reshape(L, C, H * W)
    cw2 = conv_w.reshape(C, C * 9)
    clw = cls_w.reshape(A, C)
    bbw = bbox_w.reshape(4 * A, C)
    cb = conv_b.reshape(C, 1)
    clb = cls_b.reshape(A, 1)
    bbb = bbox_b.reshape(4 * A, 1)

    scores, bbox = pl.pallas_call(
        _rpn_kernel,
        grid=(L,),
        in_specs=[
            pl.BlockSpec((1, C, H * W), lambda l: (l, 0, 0)),
            pl.BlockSpec((C, C * 9), lambda l: (0, 0)),
            pl.BlockSpec((C, 1), lambda l: (0, 0)),
            pl.BlockSpec((A, C), lambda l: (0, 0)),
            pl.BlockSpec((4 * A, C), lambda l: (0, 0)),
            pl.BlockSpec((A, 1), lambda l: (0, 0)),
            pl.BlockSpec((4 * A, 1), lambda l: (0, 0)),
        ],
        out_specs=[
            pl.BlockSpec((1, A, H, W), lambda l: (l, 0, 0, 0)),
            pl.BlockSpec((1, 4 * A, H, W), lambda l: (l, 0, 0, 0)),
        ],
        out_shape=[
            jax.ShapeDtypeStruct((L, A, H, W), jnp.float32),
            jax.ShapeDtypeStruct((L, 4 * A, H, W), jnp.float32),
        ],
        scratch_shapes=[
            pltpu.VMEM((C, S_PAD), jnp.bfloat16),
            pltpu.VMEM((9, C, C), jnp.bfloat16),
            pltpu.VMEM((NHEAD, C), jnp.bfloat16),
            pltpu.VMEM((NHEAD, 1), jnp.float32),
        ],
    )(xf, cw2, cb, clw, bbw, clb, bbb)

    return (scores, bbox)
